# initial kernel scaffold (unmeasured)
import jax
import jax.numpy as jnp
from jax import lax
from jax.experimental import pallas as pl
from jax.experimental.pallas import tpu as pltpu

N_DEV = 8
B = 128
D = 128
ROWS = B // N_DEV


def kernel(x, Win0, Wout0, Win1, Wout1, Win2, Wout2):
    def body(x_ref, win0_ref, wout0_ref, win1_ref, wout1_ref,
             win2_ref, wout2_ref, out_ref,
             acc_ref, rs_ref, part_ref, send_sems, recv_sems):
        me = lax.axis_index("i")

        def layer_partial(xv, win_ref, wout_ref):
            h = jnp.dot(xv, win_ref[:, :].astype(jnp.bfloat16),
                        preferred_element_type=jnp.float32)
            h = jnp.maximum(h, 0.0).astype(jnp.bfloat16)
            return jnp.dot(h, wout_ref[:, :].astype(jnp.bfloat16),
                           preferred_element_type=jnp.float32)

        xv = x_ref[:, :].astype(jnp.bfloat16)

        for l, (win_ref, wout_ref) in enumerate(
                [(win0_ref, wout0_ref), (win1_ref, wout1_ref)]):
            part = layer_partial(xv, win_ref, wout_ref)
            buf = l % 2
            acc_ref[buf, 0, :, :] = part.astype(jnp.bfloat16)
            rdmas = []
            for off in range(1, N_DEV):
                tgt = (me + off) % N_DEV
                rdma = pltpu.make_async_remote_copy(
                    src_ref=acc_ref.at[buf, 0],
                    dst_ref=acc_ref.at[buf, off],
                    send_sem=send_sems.at[l, off],
                    recv_sem=recv_sems.at[l, off],
                    device_id=(tgt,),
                    device_id_type=pl.DeviceIdType.MESH,
                )
                rdma.start()
                rdmas.append(rdma)
            for rdma in rdmas:
                rdma.wait_recv()
            for rdma in rdmas:
                rdma.wait_send()
            xv = jnp.sum(acc_ref[buf, :, :, :].astype(jnp.float32),
                         axis=0).astype(jnp.bfloat16)

        part = layer_partial(xv, win2_ref, wout2_ref)
        part_ref[:, :] = part.astype(jnp.bfloat16)
        rdmas = []
        for off in range(1, N_DEV):
            tgt = (me + off) % N_DEV
            rdma = pltpu.make_async_remote_copy(
                src_ref=part_ref.at[pl.ds(tgt * ROWS, ROWS), :],
                dst_ref=rs_ref.at[off],
                send_sem=send_sems.at[2, off],
                recv_sem=recv_sems.at[2, off],
                device_id=(tgt,),
                device_id_type=pl.DeviceIdType.MESH,
            )
            rdma.start()
            rdmas.append(rdma)
        for rdma in rdmas:
            rdma.wait_recv()
        for rdma in rdmas:
            rdma.wait_send()

        own = lax.dynamic_slice(part, (me * ROWS, 0), (ROWS, D))
        peers = jnp.sum(rs_ref[1:, :, :].astype(jnp.float32), axis=0)
        out_ref[:, :] = own + peers

    return pl.pallas_call(
        body,
        out_shape=jax.ShapeDtypeStruct((ROWS, D), jnp.float32),
        in_specs=[pl.BlockSpec(memory_space=pltpu.VMEM)] * 7,
        out_specs=pl.BlockSpec(memory_space=pltpu.VMEM),
        scratch_shapes=[
            pltpu.VMEM((2, N_DEV, B, D), jnp.bfloat16),
            pltpu.VMEM((N_DEV, ROWS, D), jnp.bfloat16),
            pltpu.VMEM((B, D), jnp.bfloat16),
            pltpu.SemaphoreType.DMA((3, N_DEV)),
            pltpu.SemaphoreType.DMA((3, N_DEV)),
        ],
        compiler_params=pltpu.CompilerParams(collective_id=0),
    )(x, Win0, Wout0, Win1, Wout1, Win2, Wout2)


# baseline (device time: 26290 ns/iter reference)
import jax
import jax.numpy as jnp
from jax import lax
from jax.experimental import pallas as pl
from jax.experimental.pallas import tpu as pltpu

N_DEV = 8
B = 128
D = 128
ROWS = B // N_DEV


def kernel(x, Win0, Wout0, Win1, Wout1, Win2, Wout2):
    def body(x_ref, win0_ref, wout0_ref, win1_ref, wout1_ref,
             win2_ref, wout2_ref, out_ref,
             acc_ref, rs_ref, part_ref, send_sems, recv_sems):
        me = lax.axis_index("i")

        def layer_partial(xv, win_ref, wout_ref):
            h = jnp.dot(xv, win_ref[:, :].astype(jnp.bfloat16),
                        preferred_element_type=jnp.float32)
            h = jnp.maximum(h, 0.0).astype(jnp.bfloat16)
            return jnp.dot(h, wout_ref[:, :].astype(jnp.bfloat16),
                           preferred_element_type=jnp.float32)

        xv = x_ref[:, :].astype(jnp.bfloat16)

        for l, (win_ref, wout_ref) in enumerate(
                [(win0_ref, wout0_ref), (win1_ref, wout1_ref)]):
            part = layer_partial(xv, win_ref, wout_ref)
            buf = l % 2
            acc_ref[buf, 0, :, :] = part.astype(jnp.bfloat16)
            rdmas = []
            for off in range(1, N_DEV):
                tgt = (me + off) % N_DEV
                rdma = pltpu.make_async_remote_copy(
                    src_ref=acc_ref.at[buf, 0],
                    dst_ref=acc_ref.at[buf, off],
                    send_sem=send_sems.at[l, off],
                    recv_sem=recv_sems.at[l, off],
                    device_id=(tgt,),
                    device_id_type=pl.DeviceIdType.MESH,
                )
                rdma.start()
                rdmas.append(rdma)
            for rdma in rdmas:
                rdma.wait_recv()
            for rdma in rdmas:
                rdma.wait_send()
            xv = jnp.sum(acc_ref[buf, :, :, :].astype(jnp.float32),
                         axis=0).astype(jnp.bfloat16)

        part = layer_partial(xv, win2_ref, wout2_ref)
        part_ref[:, :] = part.astype(jnp.bfloat16)
        rdmas = []
        for off in range(1, N_DEV):
            tgt = (me + off) % N_DEV
            rdma = pltpu.make_async_remote_copy(
                src_ref=part_ref.at[pl.ds(tgt * ROWS, ROWS), :],
                dst_ref=rs_ref.at[off],
                send_sem=send_sems.at[2, off],
                recv_sem=recv_sems.at[2, off],
                device_id=(tgt,),
                device_id_type=pl.DeviceIdType.MESH,
            )
            rdma.start()
            rdmas.append(rdma)
        for rdma in rdmas:
            rdma.wait_recv()
        for rdma in rdmas:
            rdma.wait_send()

        own = part_ref[pl.ds(me * ROWS, ROWS), :].astype(jnp.float32)
        peers = jnp.sum(rs_ref[1:, :, :].astype(jnp.float32), axis=0)
        out_ref[:, :] = own + peers

    return pl.pallas_call(
        body,
        out_shape=jax.ShapeDtypeStruct((ROWS, D), jnp.float32),
        in_specs=[pl.BlockSpec(memory_space=pltpu.VMEM)] * 7,
        out_specs=pl.BlockSpec(memory_space=pltpu.VMEM),
        scratch_shapes=[
            pltpu.VMEM((2, N_DEV, B, D), jnp.bfloat16),
            pltpu.VMEM((N_DEV, ROWS, D), jnp.bfloat16),
            pltpu.VMEM((B, D), jnp.bfloat16),
            pltpu.SemaphoreType.DMA((3, N_DEV)),
            pltpu.SemaphoreType.DMA((3, N_DEV)),
        ],
    )(x, Win0, Wout0, Win1, Wout1, Win2, Wout2)


# device time: 22645 ns/iter; 1.1610x vs baseline; 1.1610x over previous
import jax
import jax.numpy as jnp
from jax import lax
from jax.experimental import pallas as pl
from jax.experimental.pallas import tpu as pltpu

N_DEV = 8
B = 128
D = 128
ROWS = B // N_DEV


def kernel(x, Win0, Wout0, Win1, Wout1, Win2, Wout2):
    def body(x_ref, win0_ref, wout0_ref, win1_ref, wout1_ref,
             win2_ref, wout2_ref, out_ref,
             acc_ref, rs_ref, part_ref, send_sems, recv_sems):
        me = lax.axis_index("i")

        barrier_sem = pltpu.get_barrier_semaphore()
        for off in range(1, N_DEV):
            pl.semaphore_signal(
                barrier_sem, inc=1,
                device_id=((me + off) % N_DEV,),
                device_id_type=pl.DeviceIdType.MESH,
            )
        pl.semaphore_wait(barrier_sem, N_DEV - 1)

        def layer_partial(xv, win_ref, wout_ref):
            h = jnp.dot(xv, win_ref[:, :].astype(jnp.bfloat16),
                        preferred_element_type=jnp.float32)
            h = jnp.maximum(h, 0.0).astype(jnp.bfloat16)
            return jnp.dot(h, wout_ref[:, :].astype(jnp.bfloat16),
                           preferred_element_type=jnp.float32)

        xv = x_ref[:, :].astype(jnp.bfloat16)

        for l, (win_ref, wout_ref) in enumerate(
                [(win0_ref, wout0_ref), (win1_ref, wout1_ref)]):
            part = layer_partial(xv, win_ref, wout_ref)
            buf = l % 2
            acc_ref[buf, 0, :, :] = part.astype(jnp.bfloat16)
            rdmas = []
            for off in range(1, N_DEV):
                tgt = (me + off) % N_DEV
                rdma = pltpu.make_async_remote_copy(
                    src_ref=acc_ref.at[buf, 0],
                    dst_ref=acc_ref.at[buf, off],
                    send_sem=send_sems.at[l, off],
                    recv_sem=recv_sems.at[l, off],
                    device_id=(tgt,),
                    device_id_type=pl.DeviceIdType.MESH,
                )
                rdma.start()
                rdmas.append(rdma)
            for rdma in rdmas:
                rdma.wait_recv()
            for rdma in rdmas:
                rdma.wait_send()
            xv = jnp.sum(acc_ref[buf, :, :, :].astype(jnp.float32),
                         axis=0).astype(jnp.bfloat16)

        part = layer_partial(xv, win2_ref, wout2_ref)
        part_ref[:, :] = part.astype(jnp.bfloat16)
        rdmas = []
        for off in range(1, N_DEV):
            tgt = (me + off) % N_DEV
            rdma = pltpu.make_async_remote_copy(
                src_ref=part_ref.at[pl.ds(tgt * ROWS, ROWS), :],
                dst_ref=rs_ref.at[off],
                send_sem=send_sems.at[2, off],
                recv_sem=recv_sems.at[2, off],
                device_id=(tgt,),
                device_id_type=pl.DeviceIdType.MESH,
            )
            rdma.start()
            rdmas.append(rdma)
        for rdma in rdmas:
            rdma.wait_recv()
        for rdma in rdmas:
            rdma.wait_send()

        own = part_ref[pl.ds(me * ROWS, ROWS), :].astype(jnp.float32)
        peers = jnp.sum(rs_ref[1:, :, :].astype(jnp.float32), axis=0)
        out_ref[:, :] = own + peers

    return pl.pallas_call(
        body,
        out_shape=jax.ShapeDtypeStruct((ROWS, D), jnp.float32),
        in_specs=[pl.BlockSpec(memory_space=pltpu.VMEM)] * 7,
        out_specs=pl.BlockSpec(memory_space=pltpu.VMEM),
        scratch_shapes=[
            pltpu.VMEM((2, N_DEV, B, D), jnp.bfloat16),
            pltpu.VMEM((N_DEV, ROWS, D), jnp.bfloat16),
            pltpu.VMEM((B, D), jnp.bfloat16),
            pltpu.SemaphoreType.DMA((3, N_DEV)),
            pltpu.SemaphoreType.DMA((3, N_DEV)),
        ],
        compiler_params=pltpu.CompilerParams(collective_id=0),
    )(x, Win0, Wout0, Win1, Wout1, Win2, Wout2)
